# baseline (speedup 1.0000x reference)
import functools

import jax
import jax.numpy as jnp
from jax import lax
from jax.experimental import pallas as pl
from jax.experimental.pallas import tpu as pltpu
from jax.experimental.pallas import tpu_sc as plsc

L = 16
DIM = 64
NUM_WORKERS = 32
CHUNK = 128
_MARGIN = 1.0


def _rsqrt16(x):
    i = lax.bitcast_convert_type(x, jnp.int32)
    i = jnp.int32(0x5F3759DF) - lax.shift_right_arithmetic(i, 1)
    y = lax.bitcast_convert_type(i, jnp.float32)
    for _ in range(3):
        y = y * (1.5 - 0.5 * x * y * y)
    return y


def _make_kernel(rows_per_w, nchunk):
    mesh = plsc.VectorSubcoreMesh(core_axis_name="c", subcore_axis_name="s")

    @functools.partial(
        pl.kernel,
        mesh=mesh,
        compiler_params=pltpu.CompilerParams(
            needs_layout_passes=False, use_tc_tiling_on_sc=False),
        out_type=jax.ShapeDtypeStruct((NUM_WORKERS, L), jnp.float32),
        scratch_types=[
            pltpu.VMEM((nchunk, CHUNK), jnp.int32),
            pltpu.VMEM((nchunk, CHUNK), jnp.int32),
            pltpu.VMEM((nchunk, CHUNK), jnp.int32),
            pltpu.VMEM((rows_per_w, DIM), jnp.float32),
            pltpu.VMEM((rows_per_w, DIM), jnp.float32),
            pltpu.VMEM((rows_per_w, DIM), jnp.float32),
            pltpu.VMEM((rows_per_w,), jnp.float32),
            pltpu.VMEM((rows_per_w,), jnp.float32),
            pltpu.VMEM((L,), jnp.float32),
            pltpu.SemaphoreType.DMA,
            pltpu.SemaphoreType.DMA,
            pltpu.SemaphoreType.DMA,
        ],
    )
    def transe_sc(ph, pr, pt, nh, nr, nt, ent, rel, out,
                  hidx, ridx, tidx, hrow, rrow, trow,
                  psc, nsc, pbuf, semh, semr, semt):
        wid = lax.axis_index("s") * 2 + lax.axis_index("c")
        iota = lax.iota(jnp.int32, L)
        zf = jnp.zeros((L,), jnp.float32)
        zi = jnp.zeros((L,), jnp.int32)

        for hh, rr, tt, sc_ref in ((ph, pr, pt, psc), (nh, nr, nt, nsc)):
            pltpu.sync_copy(hh.at[wid], hidx)
            pltpu.sync_copy(rr.at[wid], ridx)
            pltpu.sync_copy(tt.at[wid], tidx)
            copies = []
            for k in range(nchunk):
                dst = pl.ds(k * CHUNK, CHUNK)
                copies.append(
                    pltpu.async_copy(ent.at[hidx.at[k]], hrow.at[dst], semh))
                copies.append(
                    pltpu.async_copy(rel.at[ridx.at[k]], rrow.at[dst], semr))
                copies.append(
                    pltpu.async_copy(ent.at[tidx.at[k]], trow.at[dst], semt))
            for c in copies:
                c.wait()

            def blk_body(b, rowv):
                def p1(j, carry):
                    sh, st, colv = carry
                    hv = plsc.load_gather(hrow, [rowv, colv])
                    tv = plsc.load_gather(trow, [rowv, colv])
                    return (sh + hv * hv, st + tv * tv, colv + 1)

                sh, st, _ = lax.fori_loop(0, DIM, p1, (zf, zf, zi))
                rih = _rsqrt16(sh)
                rit = _rsqrt16(st)

                def p2(j, carry):
                    acc, colv = carry
                    hv = plsc.load_gather(hrow, [rowv, colv])
                    rv = plsc.load_gather(rrow, [rowv, colv])
                    tv = plsc.load_gather(trow, [rowv, colv])
                    return (acc + jnp.abs(hv * rih + rv - tv * rit), colv + 1)

                acc, _ = lax.fori_loop(0, DIM, p2, (zf, zi))
                plsc.store_scatter(sc_ref, [rowv], acc)
                return rowv + L

            lax.fori_loop(0, rows_per_w // L, blk_body, iota)

        def pair_body(b, carry):
            acc, rowv = carry
            p = plsc.load_gather(psc, [rowv])
            n = plsc.load_gather(nsc, [rowv])
            return (acc + jnp.maximum(p - n + _MARGIN, 0.0), rowv + L)

        acc, _ = lax.fori_loop(0, rows_per_w // L, pair_body, (zf, iota))
        pbuf[...] = acc
        pltpu.sync_copy(pbuf, out.at[wid])

    return transe_sc


def kernel(batch_positives, batch_negatives, entity_emb, relation_emb):
    batch = batch_positives.shape[0]
    rows_per_w = batch // NUM_WORKERS
    nchunk = rows_per_w // CHUNK

    def split(b):
        cols = b.T
        return (cols[0].reshape(NUM_WORKERS, nchunk, CHUNK),
                cols[1].reshape(NUM_WORKERS, nchunk, CHUNK),
                cols[2].reshape(NUM_WORKERS, nchunk, CHUNK))

    ph, pr, pt = split(batch_positives)
    nh, nr, nt = split(batch_negatives)
    partials = _make_kernel(rows_per_w, nchunk)(
        ph, pr, pt, nh, nr, nt, entity_emb, relation_emb)
    return jnp.sum(partials) / jnp.float32(batch)


# speedup vs baseline: 1.5325x; 1.5325x over previous
import functools

import jax
import jax.numpy as jnp
from jax import lax
from jax.experimental import pallas as pl
from jax.experimental.pallas import tpu as pltpu
from jax.experimental.pallas import tpu_sc as plsc

L = 16
DIM = 64
WDIM = 128
NUM_WORKERS = 32
CHUNK = 128
_MARGIN = 1.0


def _rsqrt16(x):
    i = lax.bitcast_convert_type(x, jnp.int32)
    i = jnp.int32(0x5F3759DF) - lax.shift_right_arithmetic(i, 1)
    y = lax.bitcast_convert_type(i, jnp.float32)
    for _ in range(3):
        y = y * (1.5 - 0.5 * x * y * y)
    return y


def _make_kernel(rows_per_w, nchunk):
    mesh = plsc.VectorSubcoreMesh(core_axis_name="c", subcore_axis_name="s")

    @functools.partial(
        pl.kernel,
        mesh=mesh,
        compiler_params=pltpu.CompilerParams(needs_layout_passes=False),
        out_type=jax.ShapeDtypeStruct((NUM_WORKERS, L), jnp.float32),
        scratch_types=[
            pltpu.VMEM((nchunk, CHUNK), jnp.int32),
            pltpu.VMEM((nchunk, CHUNK), jnp.int32),
            pltpu.VMEM((nchunk, CHUNK), jnp.int32),
            pltpu.VMEM((nchunk, CHUNK), jnp.int32),
            pltpu.VMEM((nchunk, CHUNK), jnp.int32),
            pltpu.VMEM((nchunk, CHUNK), jnp.int32),
            pltpu.VMEM((nchunk, CHUNK), jnp.int32),
            pltpu.VMEM((nchunk, CHUNK), jnp.int32),
            pltpu.VMEM((nchunk, CHUNK), jnp.int32),
            pltpu.VMEM((2 * rows_per_w,), jnp.int32),
            pltpu.VMEM((2 * rows_per_w,), jnp.int32),
            pltpu.VMEM((2 * rows_per_w,), jnp.int32),
            pltpu.VMEM((CHUNK, WDIM), jnp.float32),
            pltpu.VMEM((CHUNK, WDIM), jnp.float32),
            pltpu.VMEM((CHUNK, WDIM), jnp.float32),
            pltpu.VMEM((CHUNK, WDIM), jnp.float32),
            pltpu.VMEM((CHUNK, WDIM), jnp.float32),
            pltpu.VMEM((CHUNK, WDIM), jnp.float32),
            pltpu.VMEM((rows_per_w,), jnp.float32),
            pltpu.VMEM((rows_per_w,), jnp.float32),
            pltpu.VMEM((L,), jnp.float32),
            pltpu.SemaphoreType.DMA,
            pltpu.SemaphoreType.DMA,
        ],
    )
    def transe_sc(ph, pr, pt, nh, nr, nt, ent, rel, out,
                  phidx, pridx, ptidx, nhidx, nridx, ntidx,
                  hrid, rrid, trid, hcol, rcol, tcol,
                  hA, rA, tA, hB, rB, tB,
                  psc, nsc, pbuf, semA, semB):
        wid = lax.axis_index("s") * 2 + lax.axis_index("c")
        iota = lax.iota(jnp.int32, L)
        zf = jnp.zeros((L,), jnp.float32)

        for src, dst in ((ph, phidx), (pr, pridx), (pt, ptidx),
                         (nh, nhidx), (nr, nridx), (nt, ntidx)):
            pltpu.sync_copy(src.at[wid], dst)

        for side, (hx, rx, tx) in enumerate(
                ((phidx, pridx, ptidx), (nhidx, nridx, ntidx))):
            for raw, colref in ((hx, hcol), (rx, rcol), (tx, tcol)):
                for c in range(nchunk):
                    for u in range(CHUNK // L):
                        v = raw[c, pl.ds(u * L, L)]
                        off = side * rows_per_w + c * CHUNK + u * L
                        colref[pl.ds(off, L)] = lax.shift_left(
                            jnp.bitwise_and(v, 1), 6)

        def derive_rowids(hx, rx, tx):
            for raw, rid in ((hx, hrid), (rx, rrid), (tx, trid)):
                for c in range(nchunk):
                    for u in range(CHUNK // L):
                        s = pl.ds(u * L, L)
                        rid[c, s] = lax.shift_right_logical(raw[c, s], 1)

        def fire(seg, bufs, sem):
            d = pl.ds(0, CHUNK)
            return [
                pltpu.async_copy(ent.at[hrid.at[seg]], bufs[0].at[d], sem),
                pltpu.async_copy(rel.at[rrid.at[seg]], bufs[1].at[d], sem),
                pltpu.async_copy(ent.at[trid.at[seg]], bufs[2].at[d], sem),
            ]

        def compute(bufs, scref, base):
            hrow, rrow, trow = bufs

            def blk(b, rowv):
                gidxv = rowv + base
                hc = plsc.load_gather(hcol, [gidxv])
                rc = plsc.load_gather(rcol, [gidxv])
                tc = plsc.load_gather(tcol, [gidxv])

                def p1(u, carry):
                    sh, st, skv = carry
                    for _ in range(16):
                        hv = plsc.load_gather(hrow, [rowv, hc + skv])
                        tv = plsc.load_gather(trow, [rowv, tc + skv])
                        sh = sh + hv * hv
                        st = st + tv * tv
                        skv = (skv + 1) & (DIM - 1)
                    return sh, st, skv

                sh, st, _ = lax.fori_loop(0, DIM // 16, p1, (zf, zf, iota))
                rih = _rsqrt16(sh)
                rit = _rsqrt16(st)

                def p2(u, carry):
                    acc, skv = carry
                    for _ in range(16):
                        hv = plsc.load_gather(hrow, [rowv, hc + skv])
                        rv = plsc.load_gather(rrow, [rowv, rc + skv])
                        tv = plsc.load_gather(trow, [rowv, tc + skv])
                        acc = acc + jnp.abs(hv * rih + rv - tv * rit)
                        skv = (skv + 1) & (DIM - 1)
                    return acc, skv

                acc, _ = lax.fori_loop(0, DIM // 16, p2, (zf, iota))
                plsc.store_scatter(scref, [rowv + (base % rows_per_w)], acc)
                return rowv + L

            lax.fori_loop(0, CHUNK // L, blk, iota)

        A = (hA, rA, tA)
        B = (hB, rB, tB)
        segplan = [(side, seg) for side in range(2) for seg in range(nchunk)]
        derive_rowids(phidx, pridx, ptidx)
        pending = [fire(0, A, semA), fire(1, B, semB)]
        fired_side = 0
        for i, (side, seg) in enumerate(segplan):
            bufs, sem = (A, semA) if i % 2 == 0 else (B, semB)
            for c in pending.pop(0):
                c.wait()
            scref = psc if side == 0 else nsc
            compute(bufs, scref, side * rows_per_w + seg * CHUNK)
            j = i + 2
            if j < len(segplan):
                nside, nseg = segplan[j]
                if nside == 1 and fired_side == 0:
                    derive_rowids(nhidx, nridx, ntidx)
                fired_side = nside
                pending.append(fire(nseg, bufs, sem))

        accv = zf
        for b in range(rows_per_w // L):
            p = psc[pl.ds(b * L, L)]
            n = nsc[pl.ds(b * L, L)]
            accv = accv + jnp.maximum(p - n + _MARGIN, 0.0)
        pbuf[...] = accv
        pltpu.sync_copy(pbuf, out.at[wid])

    return transe_sc


def kernel(batch_positives, batch_negatives, entity_emb, relation_emb):
    batch = batch_positives.shape[0]
    rows_per_w = batch // NUM_WORKERS
    nchunk = rows_per_w // CHUNK

    def split(b):
        cols = b.T
        return (cols[0].reshape(NUM_WORKERS, nchunk, CHUNK),
                cols[1].reshape(NUM_WORKERS, nchunk, CHUNK),
                cols[2].reshape(NUM_WORKERS, nchunk, CHUNK))

    ph, pr, pt = split(batch_positives)
    nh, nr, nt = split(batch_negatives)
    ent2 = entity_emb.reshape(-1, WDIM)
    rel2 = relation_emb.reshape(-1, WDIM)
    partials = _make_kernel(rows_per_w, nchunk)(
        ph, pr, pt, nh, nr, nt, ent2, rel2)
    return jnp.sum(partials) / jnp.float32(batch)


# speedup vs baseline: 1.8284x; 1.1931x over previous
import functools

import jax
import jax.numpy as jnp
from jax import lax
from jax.experimental import pallas as pl
from jax.experimental.pallas import tpu as pltpu
from jax.experimental.pallas import tpu_sc as plsc

L = 16
DIM = 64
NUM_WORKERS = 32
CHUNK = 128
HALF = 256
_MARGIN = 1.0


def _rsqrt16(x):
    i = lax.bitcast_convert_type(x, jnp.int32)
    i = jnp.int32(0x5F3759DF) - lax.shift_right_arithmetic(i, 1)
    y = lax.bitcast_convert_type(i, jnp.float32)
    for _ in range(3):
        y = y * (1.5 - 0.5 * x * y * y)
    return y


def _make_kernel(rows_per_w, nchunk):
    mesh = plsc.VectorSubcoreMesh(core_axis_name="c", subcore_axis_name="s")

    @functools.partial(
        pl.kernel,
        mesh=mesh,
        compiler_params=pltpu.CompilerParams(
            needs_layout_passes=False, use_tc_tiling_on_sc=False),
        out_type=jax.ShapeDtypeStruct((NUM_WORKERS, L), jnp.float32),
        scratch_types=[
            pltpu.VMEM((3 * rows_per_w,), jnp.int32),
            pltpu.VMEM((3 * rows_per_w,), jnp.int32),
            pltpu.VMEM((nchunk, CHUNK), jnp.int32),
            pltpu.VMEM((nchunk, CHUNK), jnp.int32),
            pltpu.VMEM((nchunk, CHUNK), jnp.int32),
            pltpu.VMEM((nchunk, CHUNK), jnp.int32),
            pltpu.VMEM((nchunk, CHUNK), jnp.int32),
            pltpu.VMEM((nchunk, CHUNK), jnp.int32),
            pltpu.VMEM((HALF, DIM), jnp.float32),
            pltpu.VMEM((HALF, DIM), jnp.float32),
            pltpu.VMEM((HALF, DIM), jnp.float32),
            pltpu.VMEM((HALF, DIM), jnp.float32),
            pltpu.VMEM((HALF, DIM), jnp.float32),
            pltpu.VMEM((HALF, DIM), jnp.float32),
            pltpu.VMEM((rows_per_w,), jnp.float32),
            pltpu.VMEM((rows_per_w,), jnp.float32),
            pltpu.VMEM((L,), jnp.float32),
            pltpu.SemaphoreType.DMA,
            pltpu.SemaphoreType.DMA,
        ],
    )
    def transe_sc(pflat, nflat, ent, rel, out,
                  rawp, rawn,
                  phidx, pridx, ptidx, nhidx, nridx, ntidx,
                  hA, rA, tA, hB, rB, tB,
                  psc, nsc, pbuf, semA, semB):
        wid = lax.axis_index("s") * 2 + lax.axis_index("c")
        iota = lax.iota(jnp.int32, L)
        zf = jnp.zeros((L,), jnp.float32)
        zi = jnp.zeros((L,), jnp.int32)

        pltpu.sync_copy(pflat.at[pl.ds(wid * (3 * rows_per_w), 3 * rows_per_w)],
                        rawp)
        pltpu.sync_copy(nflat.at[pl.ds(wid * (3 * rows_per_w), 3 * rows_per_w)],
                        rawn)
        stride3 = iota * 3
        for raw, (hx, rx, tx) in ((rawp, (phidx, pridx, ptidx)),
                                  (rawn, (nhidx, nridx, ntidx))):
            for g in range(rows_per_w // L):
                idxv = stride3 + (g * 3 * L)
                c, u = divmod(g, CHUNK // L)
                s = pl.ds(u * L, L)
                hx[c, s] = plsc.load_gather(raw, [idxv])
                rx[c, s] = plsc.load_gather(raw, [idxv + 1])
                tx[c, s] = plsc.load_gather(raw, [idxv + 2])

        def fire(hx, rx, tx, half, bufs, sem):
            cps = []
            for k in range(HALF // CHUNK):
                c = half * (HALF // CHUNK) + k
                d = pl.ds(k * CHUNK, CHUNK)
                cps.append(pltpu.async_copy(ent.at[hx.at[c]], bufs[0].at[d], sem))
                cps.append(pltpu.async_copy(rel.at[rx.at[c]], bufs[1].at[d], sem))
                cps.append(pltpu.async_copy(ent.at[tx.at[c]], bufs[2].at[d], sem))
            return cps

        def compute(bufs, scref, base):
            hrow, rrow, trow = bufs

            def blk(b, rowv):
                def p1(u, carry):
                    sh, st, colv = carry
                    for _ in range(16):
                        hv = plsc.load_gather(hrow, [rowv, colv])
                        tv = plsc.load_gather(trow, [rowv, colv])
                        sh = sh + hv * hv
                        st = st + tv * tv
                        colv = (colv + 1) & (DIM - 1)
                    return sh, st, colv

                sh, st, _ = lax.fori_loop(0, DIM // 16, p1, (zf, zf, iota))
                rih = _rsqrt16(sh)
                rit = _rsqrt16(st)

                def p2(u, carry):
                    acc, colv = carry
                    for _ in range(16):
                        hv = plsc.load_gather(hrow, [rowv, colv])
                        rv = plsc.load_gather(rrow, [rowv, colv])
                        tv = plsc.load_gather(trow, [rowv, colv])
                        acc = acc + jnp.abs(hv * rih + rv - tv * rit)
                        colv = (colv + 1) & (DIM - 1)
                    return acc, colv

                acc, _ = lax.fori_loop(0, DIM // 16, p2, (zf, iota))
                plsc.store_scatter(scref, [rowv + base], acc)
                return rowv + L

            lax.fori_loop(0, HALF // L, blk, iota)

        A = (hA, rA, tA)
        B = (hB, rB, tB)
        pending = [fire(phidx, pridx, ptidx, 0, A, semA),
                   fire(phidx, pridx, ptidx, 1, B, semB)]
        plan = [
            (A, psc, 0, (nhidx, nridx, ntidx, 0, A, semA)),
            (B, psc, HALF, (nhidx, nridx, ntidx, 1, B, semB)),
            (A, nsc, 0, None),
            (B, nsc, HALF, None),
        ]
        for bufs, scref, base, refire in plan:
            for c in pending.pop(0):
                c.wait()
            compute(bufs, scref, base)
            if refire is not None:
                pending.append(fire(*refire))

        accv = zf
        for b in range(rows_per_w // L):
            p = psc[pl.ds(b * L, L)]
            n = nsc[pl.ds(b * L, L)]
            accv = accv + jnp.maximum(p - n + _MARGIN, 0.0)
        pbuf[...] = accv
        pltpu.sync_copy(pbuf, out.at[wid])

    return transe_sc


def kernel(batch_positives, batch_negatives, entity_emb, relation_emb):
    batch = batch_positives.shape[0]
    rows_per_w = batch // NUM_WORKERS
    nchunk = rows_per_w // CHUNK

    partials = _make_kernel(rows_per_w, nchunk)(
        batch_positives.reshape(-1), batch_negatives.reshape(-1),
        entity_emb, relation_emb)
    return jnp.sum(partials) / jnp.float32(batch)
